# baseline (speedup 1.0000x reference)
import functools

import jax
import jax.numpy as jnp
import numpy as np
from jax import lax
from jax.experimental import pallas as pl
from jax.experimental.pallas import tpu as pltpu
from jax.experimental.pallas import tpu_sc as plsc

N = 10000
E = 320000
H = 128
RBF = 16
CUTOFF = 5.0
EPS = 1e-8

NC = 2
NS = 16
WIN = 128
NWIN = E // WIN
BE = 3200
BN = 2000


def _node_tables_body(xs_ref, xv_ref, w1_ref, b1_ref, w2_ref, b2_ref,
                      out_ref):
    h = jnp.dot(xs_ref[...], w1_ref[...], preferred_element_type=jnp.float32)
    h = h + b1_ref[...]
    h = h * jax.nn.sigmoid(h)
    x = jnp.dot(h, w2_ref[...], preferred_element_type=jnp.float32) + b2_ref[...]
    xb = x.astype(jnp.bfloat16).astype(jnp.float32)
    xvb = xv_ref[...].astype(jnp.bfloat16).astype(jnp.float32)
    xi = jax.lax.bitcast_convert_type(xb, jnp.int32)
    xvi = jax.lax.bitcast_convert_type(xvb, jnp.int32)
    out_ref[...] = jax.lax.shift_right_logical(xi, 16) | (xvi & jnp.int32(-65536))


def _edge_filter_body(ea_ref, ew_ref, wf_ref, bf_ref, out_ref):
    ew = ew_ref[...]
    c = 0.5 * (jnp.cos(ew * (np.pi / CUTOFF)) + 1.0)
    c = c * (ew < CUTOFF).astype(jnp.float32)
    w = jnp.dot(ea_ref[...], wf_ref[...], preferred_element_type=jnp.float32)
    out_ref[...] = (w + bf_ref[...]) * c


def _message_body(w_ref, xxvj_ref, nd_ref, out_ref):
    w = w_ref[...]
    packed = xxvj_ref[...]
    xj = jax.lax.bitcast_convert_type(packed << 16, jnp.float32)
    xvj = jax.lax.bitcast_convert_type(packed & jnp.int32(-65536), jnp.float32)
    out_ref[0] = w[:, :H] * xj[:, :H]
    dmu_r = w[:, H:2 * H] * xj[:, H:2 * H]
    dmu_mu = w[:, 2 * H:] * xj[:, 2 * H:]
    for c in range(3):
        out_ref[1 + c] = (
            dmu_r * nd_ref[:, c:c + 1] + dmu_mu * xvj[:, c * H:(c + 1) * H]
        )


def _mixing_body(xs_ref, xv_ref, agg_ref, wmix_ref, wm1_ref, bm1_ref,
                 wm2_ref, bm2_ref, s_out_ref, v_out_ref):
    s = xs_ref[...] + agg_ref[0]
    v = [xv_ref[:, c * H:(c + 1) * H] + agg_ref[1 + c] for c in range(3)]
    mm = [jnp.dot(v[c], wmix_ref[...], preferred_element_type=jnp.float32)
          for c in range(3)]
    mu_v = [m[:, :H] for m in mm]
    mu_w = [m[:, H:] for m in mm]
    mu_vn = jnp.sqrt(mu_v[0] ** 2 + mu_v[1] ** 2 + mu_v[2] ** 2 + EPS)
    ctx = jnp.concatenate([s, mu_vn], axis=-1)
    h = jnp.dot(ctx, wm1_ref[...], preferred_element_type=jnp.float32) + bm1_ref[...]
    h = h * jax.nn.sigmoid(h)
    xc = jnp.dot(h, wm2_ref[...], preferred_element_type=jnp.float32) + bm2_ref[...]
    dq_i = xc[:, :H]
    dmu_i = xc[:, H:2 * H]
    dqmu_i = xc[:, 2 * H:]
    sum_vw = mu_v[0] * mu_w[0] + mu_v[1] * mu_w[1] + mu_v[2] * mu_w[2]
    s_out_ref[...] = s + dq_i + dqmu_i * sum_vw
    v_out_ref[...] = jnp.concatenate(
        [v[c] + dmu_i * mu_w[c] for c in range(3)], axis=-1)


def _node_tables(xs, xv, w1, b1, w2, b2):
    return pl.pallas_call(
        _node_tables_body,
        grid=(N // BN,),
        in_specs=[
            pl.BlockSpec((BN, H), lambda i: (i, 0)),
            pl.BlockSpec((BN, 3 * H), lambda i: (i, 0)),
            pl.BlockSpec((H, H), lambda i: (0, 0)),
            pl.BlockSpec((1, H), lambda i: (0, 0)),
            pl.BlockSpec((H, 3 * H), lambda i: (0, 0)),
            pl.BlockSpec((1, 3 * H), lambda i: (0, 0)),
        ],
        out_specs=pl.BlockSpec((BN, 3 * H), lambda i: (i, 0)),
        out_shape=jax.ShapeDtypeStruct((N, 3 * H), jnp.int32),
    )(xs, xv, w1, b1, w2, b2)


def _edge_filter(ea, ew, wf, bf):
    return pl.pallas_call(
        _edge_filter_body,
        grid=(E // BE,),
        in_specs=[
            pl.BlockSpec((BE, RBF), lambda i: (i, 0)),
            pl.BlockSpec((BE, 1), lambda i: (i, 0)),
            pl.BlockSpec((RBF, 3 * H), lambda i: (0, 0)),
            pl.BlockSpec((1, 3 * H), lambda i: (0, 0)),
        ],
        out_specs=pl.BlockSpec((BE, 3 * H), lambda i: (i, 0)),
        out_shape=jax.ShapeDtypeStruct((E, 3 * H), jnp.float32),
    )(ea, ew, wf, bf)


def _messages(w, xxvj, nd):
    return pl.pallas_call(
        _message_body,
        grid=(E // BE,),
        in_specs=[
            pl.BlockSpec((BE, 3 * H), lambda i: (i, 0)),
            pl.BlockSpec((BE, 3 * H), lambda i: (i, 0)),
            pl.BlockSpec((BE, 3), lambda i: (i, 0)),
        ],
        out_specs=pl.BlockSpec((4, BE, H), lambda i: (0, i, 0)),
        out_shape=jax.ShapeDtypeStruct((4, E, H), jnp.float32),
    )(w, xxvj, nd)


def _mixing(xs, xv, agg, wmix, wm1, bm1, wm2, bm2):
    return pl.pallas_call(
        _mixing_body,
        grid=(N // BN,),
        in_specs=[
            pl.BlockSpec((BN, H), lambda i: (i, 0)),
            pl.BlockSpec((BN, 3 * H), lambda i: (i, 0)),
            pl.BlockSpec((4, BN, H), lambda i: (0, i, 0)),
            pl.BlockSpec((H, 2 * H), lambda i: (0, 0)),
            pl.BlockSpec((2 * H, H), lambda i: (0, 0)),
            pl.BlockSpec((1, H), lambda i: (0, 0)),
            pl.BlockSpec((H, 3 * H), lambda i: (0, 0)),
            pl.BlockSpec((1, 3 * H), lambda i: (0, 0)),
        ],
        out_specs=[
            pl.BlockSpec((BN, H), lambda i: (i, 0)),
            pl.BlockSpec((BN, 3 * H), lambda i: (i, 0)),
        ],
        out_shape=[
            jax.ShapeDtypeStruct((N, H), jnp.float32),
            jax.ShapeDtypeStruct((N, 3 * H), jnp.float32),
        ],
    )(xs, xv, agg, wmix, wm1, bm1, wm2, bm2)


def _sc_mesh():
    return plsc.VectorSubcoreMesh(
        core_axis_name="c", subcore_axis_name="s", num_cores=NC, num_subcores=NS)


def _sc_gather_body(tab_hbm, src_hbm, out_hbm,
                    idx0, idx1, gb0, gb1, si0, si1, sg0, sg1, sw0, sw1):
    wid = lax.axis_index("s") * NC + lax.axis_index("c")
    nw = NC * NS
    idxb = (idx0, idx1)
    gbuf = (gb0, gb1)
    si = (si0, si1)
    sg = (sg0, sg1)
    sw = (sw0, sw1)

    def start_idx(w, s):
        pltpu.async_copy(src_hbm.at[pl.ds(w * WIN, WIN)], idxb[s], si[s])

    def wait_idx(s):
        pltpu.make_async_copy(src_hbm.at[pl.ds(0, WIN)], idxb[s], si[s]).wait()

    def start_g(s):
        pltpu.async_copy(tab_hbm.at[idxb[s]], gbuf[s], sg[s])

    def wait_g(s):
        pltpu.make_async_copy(tab_hbm.at[idxb[s]], gbuf[s], sg[s]).wait()

    def start_wr(w, s):
        pltpu.async_copy(gbuf[s], out_hbm.at[pl.ds(w * WIN, WIN)], sw[s])

    def wait_wr(s):
        pltpu.make_async_copy(gbuf[s], out_hbm.at[pl.ds(0, WIN)], sw[s]).wait()

    start_idx(wid, 0)
    start_idx(wid + nw, 1)
    wait_idx(0)
    start_g(0)

    @pl.loop(0, 40)
    def _(p):
        for r in range(2):
            k = p * 2 + r
            w = wid + nw * k
            s = r
            s1 = (r + 1) % 2

            @pl.when(w < NWIN)
            def _():
                wait_g(s)

            @pl.when(jnp.logical_and(k >= 1, w - nw < NWIN))
            def _():
                wait_wr(s1)

            @pl.when(w + nw < NWIN)
            def _():
                wait_idx(s1)
                start_g(s1)

            @pl.when(w < NWIN)
            def _():
                start_wr(w, s)

            @pl.when(w + 2 * nw < NWIN)
            def _():
                start_idx(w + 2 * nw, s)


def _sc_gather(tab, srcidx):
    k = pl.kernel(
        _sc_gather_body,
        out_type=jax.ShapeDtypeStruct((E, 3 * H), jnp.int32),
        mesh=_sc_mesh(),
        scratch_types=(
            [pltpu.VMEM((WIN,), jnp.int32)] * 2
            + [pltpu.VMEM((WIN, 3 * H), jnp.int32)] * 2
            + [pltpu.SemaphoreType.DMA] * 6
        ),
    )
    return k(tab, srcidx)


def _sc_scatter_body(msg_hbm, dst_hbm, zeros_hbm, out_hbm, idxb, msgb, acc):
    cid = lax.axis_index("c")
    sid = lax.axis_index("s")
    rows = N // NS
    iters = (NWIN + NS - 1) // NS

    for j in range(2):
        chunk = cid * 2 + j

        @pl.when(sid == 0)
        def _():
            pltpu.sync_copy(zeros_hbm, acc)

        plsc.subcore_barrier()

        @pl.loop(0, iters)
        def _(k):
            w = sid + NS * k

            @pl.when(w < NWIN)
            def _():
                base = w * WIN
                pltpu.sync_copy(dst_hbm.at[pl.ds(base, WIN)], idxb)
                pltpu.sync_copy(msg_hbm.at[chunk].at[pl.ds(base, WIN)], msgb)
                pltpu.sync_copy(msgb, acc.at[idxb], add=True)

        plsc.subcore_barrier()

        @pl.when(sid < NS - 1)
        def _():
            pltpu.sync_copy(
                acc.at[pl.ds(sid * 640, 640)],
                out_hbm.at[chunk].at[pl.ds(sid * 640, 640)])

        @pl.when(sid == NS - 1)
        def _():
            pltpu.sync_copy(
                acc.at[pl.ds(9600, N - 9600)],
                out_hbm.at[chunk].at[pl.ds(9600, N - 9600)])

        plsc.subcore_barrier()


def _sc_scatter(msg, dst, zeros):
    k = pl.kernel(
        _sc_scatter_body,
        out_type=jax.ShapeDtypeStruct((4, N, H), jnp.float32),
        mesh=_sc_mesh(),
        scratch_types=[
            pltpu.VMEM((WIN,), jnp.int32),
            pltpu.VMEM((WIN, H), jnp.float32),
            pltpu.VMEM_SHARED((N, H), jnp.float32),
        ],
    )
    return k(msg, dst, zeros)


def kernel(scalar_node_features, vector_node_features, normdir, edge_index,
           edge_weight, edge_attr, W1, b1, W2, b2, Wf, bf, Wm1, bm1, Wm2, bm2,
           Wmix):
    xs = scalar_node_features[:, 0, :]
    xv = vector_node_features.reshape(N, 3 * H)
    src = edge_index[0]
    dst = edge_index[1]
    ew = edge_weight.reshape(E, 1)

    xxv = _node_tables(xs, xv, W1, b1.reshape(1, H), W2, b2.reshape(1, 3 * H))
    w_filt = _edge_filter(edge_attr, ew, Wf, bf.reshape(1, 3 * H))
    xxvj = _sc_gather(xxv, src)
    msg = _messages(w_filt, xxvj, normdir)
    zeros = jnp.zeros((N, H), dtype=jnp.float32)
    agg = _sc_scatter(msg, dst, zeros)
    s_out, v_out = _mixing(xs, xv, agg, Wmix, Wm1, bm1.reshape(1, H),
                           Wm2, bm2.reshape(1, 3 * H))
    return s_out.reshape(N, 1, H), v_out.reshape(N, 3, H)


# speedup vs baseline: 1.0293x; 1.0293x over previous
import functools

import jax
import jax.numpy as jnp
import numpy as np
from jax import lax
from jax.experimental import pallas as pl
from jax.experimental.pallas import tpu as pltpu
from jax.experimental.pallas import tpu_sc as plsc

N = 10000
E = 320000
H = 128
RBF = 16
CUTOFF = 5.0
EPS = 1e-8

NC = 2
NS = 16
WIN = 128
NWIN = E // WIN
BE = 3200
E2 = E // 2
NWIN2 = E2 // WIN
BN = 2000


def _node_tables_body(xs_ref, xv_ref, w1_ref, b1_ref, w2_ref, b2_ref,
                      out_ref):
    h = jnp.dot(xs_ref[...], w1_ref[...], preferred_element_type=jnp.float32)
    h = h + b1_ref[...]
    h = h * jax.nn.sigmoid(h)
    x = jnp.dot(h, w2_ref[...], preferred_element_type=jnp.float32) + b2_ref[...]
    xb = x.astype(jnp.bfloat16).astype(jnp.float32)
    xvb = xv_ref[...].astype(jnp.bfloat16).astype(jnp.float32)
    xi = jax.lax.bitcast_convert_type(xb, jnp.int32)
    xvi = jax.lax.bitcast_convert_type(xvb, jnp.int32)
    out_ref[...] = jax.lax.shift_right_logical(xi, 16) | (xvi & jnp.int32(-65536))


def _edge_filter_body(ea_ref, ew_ref, wf_ref, bf_ref, out_ref):
    ew = ew_ref[...]
    c = 0.5 * (jnp.cos(ew * (np.pi / CUTOFF)) + 1.0)
    c = c * (ew < CUTOFF).astype(jnp.float32)
    w = jnp.dot(ea_ref[...], wf_ref[...], preferred_element_type=jnp.float32)
    out_ref[...] = (w + bf_ref[...]) * c


def _message_body(w_ref, xxvj_ref, nd_ref, out_ref):
    w = w_ref[...]
    packed = xxvj_ref[...]
    xj = jax.lax.bitcast_convert_type(packed << 16, jnp.float32)
    xvj = jax.lax.bitcast_convert_type(packed & jnp.int32(-65536), jnp.float32)
    out_ref[0] = w[:, :H] * xj[:, :H]
    dmu_r = w[:, H:2 * H] * xj[:, H:2 * H]
    dmu_mu = w[:, 2 * H:] * xj[:, 2 * H:]
    for c in range(3):
        out_ref[1 + c] = (
            dmu_r * nd_ref[:, c:c + 1] + dmu_mu * xvj[:, c * H:(c + 1) * H]
        )


def _mixing_body(xs_ref, xv_ref, agg_ref, wmix_ref, wm1_ref, bm1_ref,
                 wm2_ref, bm2_ref, s_out_ref, v_out_ref):
    s = xs_ref[...] + agg_ref[0]
    v = [xv_ref[:, c * H:(c + 1) * H] + agg_ref[1 + c] for c in range(3)]
    mm = [jnp.dot(v[c], wmix_ref[...], preferred_element_type=jnp.float32)
          for c in range(3)]
    mu_v = [m[:, :H] for m in mm]
    mu_w = [m[:, H:] for m in mm]
    mu_vn = jnp.sqrt(mu_v[0] ** 2 + mu_v[1] ** 2 + mu_v[2] ** 2 + EPS)
    ctx = jnp.concatenate([s, mu_vn], axis=-1)
    h = jnp.dot(ctx, wm1_ref[...], preferred_element_type=jnp.float32) + bm1_ref[...]
    h = h * jax.nn.sigmoid(h)
    xc = jnp.dot(h, wm2_ref[...], preferred_element_type=jnp.float32) + bm2_ref[...]
    dq_i = xc[:, :H]
    dmu_i = xc[:, H:2 * H]
    dqmu_i = xc[:, 2 * H:]
    sum_vw = mu_v[0] * mu_w[0] + mu_v[1] * mu_w[1] + mu_v[2] * mu_w[2]
    s_out_ref[...] = s + dq_i + dqmu_i * sum_vw
    v_out_ref[...] = jnp.concatenate(
        [v[c] + dmu_i * mu_w[c] for c in range(3)], axis=-1)


def _node_tables(xs, xv, w1, b1, w2, b2):
    return pl.pallas_call(
        _node_tables_body,
        grid=(N // BN,),
        in_specs=[
            pl.BlockSpec((BN, H), lambda i: (i, 0)),
            pl.BlockSpec((BN, 3 * H), lambda i: (i, 0)),
            pl.BlockSpec((H, H), lambda i: (0, 0)),
            pl.BlockSpec((1, H), lambda i: (0, 0)),
            pl.BlockSpec((H, 3 * H), lambda i: (0, 0)),
            pl.BlockSpec((1, 3 * H), lambda i: (0, 0)),
        ],
        out_specs=pl.BlockSpec((BN, 3 * H), lambda i: (i, 0)),
        out_shape=jax.ShapeDtypeStruct((N, 3 * H), jnp.int32),
    )(xs, xv, w1, b1, w2, b2)


def _edge_filter(ea, ew, wf, bf):
    return pl.pallas_call(
        _edge_filter_body,
        grid=(E // BE,),
        in_specs=[
            pl.BlockSpec((BE, RBF), lambda i: (i, 0)),
            pl.BlockSpec((BE, 1), lambda i: (i, 0)),
            pl.BlockSpec((RBF, 3 * H), lambda i: (0, 0)),
            pl.BlockSpec((1, 3 * H), lambda i: (0, 0)),
        ],
        out_specs=pl.BlockSpec((BE, 3 * H), lambda i: (i, 0)),
        out_shape=jax.ShapeDtypeStruct((E, 3 * H), jnp.float32),
    )(ea, ew, wf, bf)


def _messages(w, xxvj, nd, half):
    off = half * (E2 // BE)
    return pl.pallas_call(
        _message_body,
        grid=(E2 // BE,),
        in_specs=[
            pl.BlockSpec((BE, 3 * H), lambda i: (i + off, 0)),
            pl.BlockSpec((BE, 3 * H), lambda i: (i, 0)),
            pl.BlockSpec((BE, 3), lambda i: (i + off, 0)),
        ],
        out_specs=pl.BlockSpec((4, BE, H), lambda i: (0, i, 0)),
        out_shape=jax.ShapeDtypeStruct((4, E2, H), jnp.float32),
    )(w, xxvj, nd)


def _mixing(xs, xv, agg, wmix, wm1, bm1, wm2, bm2):
    return pl.pallas_call(
        _mixing_body,
        grid=(N // BN,),
        in_specs=[
            pl.BlockSpec((BN, H), lambda i: (i, 0)),
            pl.BlockSpec((BN, 3 * H), lambda i: (i, 0)),
            pl.BlockSpec((4, BN, H), lambda i: (0, i, 0)),
            pl.BlockSpec((H, 2 * H), lambda i: (0, 0)),
            pl.BlockSpec((2 * H, H), lambda i: (0, 0)),
            pl.BlockSpec((1, H), lambda i: (0, 0)),
            pl.BlockSpec((H, 3 * H), lambda i: (0, 0)),
            pl.BlockSpec((1, 3 * H), lambda i: (0, 0)),
        ],
        out_specs=[
            pl.BlockSpec((BN, H), lambda i: (i, 0)),
            pl.BlockSpec((BN, 3 * H), lambda i: (i, 0)),
        ],
        out_shape=[
            jax.ShapeDtypeStruct((N, H), jnp.float32),
            jax.ShapeDtypeStruct((N, 3 * H), jnp.float32),
        ],
    )(xs, xv, agg, wmix, wm1, bm1, wm2, bm2)


def _sc_mesh():
    return plsc.VectorSubcoreMesh(
        core_axis_name="c", subcore_axis_name="s", num_cores=NC, num_subcores=NS)


def _sc_gather_body(tab_hbm, src_hbm, out_hbm,
                    idx0, idx1, gb0, gb1, si0, si1, sg0, sg1, sw0, sw1):
    wid = lax.axis_index("s") * NC + lax.axis_index("c")
    nw = NC * NS
    idxb = (idx0, idx1)
    gbuf = (gb0, gb1)
    si = (si0, si1)
    sg = (sg0, sg1)
    sw = (sw0, sw1)

    def start_idx(w, s):
        pltpu.async_copy(src_hbm.at[pl.ds(w * WIN, WIN)], idxb[s], si[s])

    def wait_idx(s):
        pltpu.make_async_copy(src_hbm.at[pl.ds(0, WIN)], idxb[s], si[s]).wait()

    def start_g(s):
        pltpu.async_copy(tab_hbm.at[idxb[s]], gbuf[s], sg[s])

    def wait_g(s):
        pltpu.make_async_copy(tab_hbm.at[idxb[s]], gbuf[s], sg[s]).wait()

    def start_wr(w, s):
        pltpu.async_copy(gbuf[s], out_hbm.at[pl.ds(w * WIN, WIN)], sw[s])

    def wait_wr(s):
        pltpu.make_async_copy(gbuf[s], out_hbm.at[pl.ds(0, WIN)], sw[s]).wait()

    start_idx(wid, 0)
    start_idx(wid + nw, 1)
    wait_idx(0)
    start_g(0)

    @pl.loop(0, NWIN2 // (2 * NC * NS) + 2)
    def _(p):
        for r in range(2):
            k = p * 2 + r
            w = wid + nw * k
            s = r
            s1 = (r + 1) % 2

            @pl.when(w < NWIN2)
            def _():
                wait_g(s)

            @pl.when(jnp.logical_and(k >= 1, w - nw < NWIN2))
            def _():
                wait_wr(s1)

            @pl.when(w + nw < NWIN2)
            def _():
                wait_idx(s1)
                start_g(s1)

            @pl.when(w < NWIN2)
            def _():
                start_wr(w, s)

            @pl.when(w + 2 * nw < NWIN2)
            def _():
                start_idx(w + 2 * nw, s)


def _sc_gather(tab, srcidx):
    k = pl.kernel(
        _sc_gather_body,
        out_type=jax.ShapeDtypeStruct((E2, 3 * H), jnp.int32),
        mesh=_sc_mesh(),
        scratch_types=(
            [pltpu.VMEM((WIN,), jnp.int32)] * 2
            + [pltpu.VMEM((WIN, 3 * H), jnp.int32)] * 2
            + [pltpu.SemaphoreType.DMA] * 6
        ),
    )
    return k(tab, srcidx)


def _sc_scatter_body(msg_hbm, dst_hbm, init_hbm, out_hbm, idxb, msgb, acc):
    cid = lax.axis_index("c")
    sid = lax.axis_index("s")
    iters = (NWIN2 + NS - 1) // NS

    for j in range(2):
        chunk = cid * 2 + j

        @pl.when(sid == 0)
        def _():
            pltpu.sync_copy(init_hbm.at[chunk], acc)

        plsc.subcore_barrier()

        @pl.loop(0, iters)
        def _(k):
            w = sid + NS * k

            @pl.when(w < NWIN2)
            def _():
                base = w * WIN
                pltpu.sync_copy(dst_hbm.at[pl.ds(base, WIN)], idxb)
                pltpu.sync_copy(msg_hbm.at[chunk].at[pl.ds(base, WIN)], msgb)
                pltpu.sync_copy(msgb, acc.at[idxb], add=True)

        plsc.subcore_barrier()

        @pl.when(sid < NS - 1)
        def _():
            pltpu.sync_copy(
                acc.at[pl.ds(sid * 640, 640)],
                out_hbm.at[chunk].at[pl.ds(sid * 640, 640)])

        @pl.when(sid == NS - 1)
        def _():
            pltpu.sync_copy(
                acc.at[pl.ds(9600, N - 9600)],
                out_hbm.at[chunk].at[pl.ds(9600, N - 9600)])

        plsc.subcore_barrier()


def _sc_scatter(msg, dst, init):
    k = pl.kernel(
        _sc_scatter_body,
        out_type=jax.ShapeDtypeStruct((4, N, H), jnp.float32),
        mesh=_sc_mesh(),
        scratch_types=[
            pltpu.VMEM((WIN,), jnp.int32),
            pltpu.VMEM((WIN, H), jnp.float32),
            pltpu.VMEM_SHARED((N, H), jnp.float32),
        ],
    )
    return k(msg, dst, init)


def kernel(scalar_node_features, vector_node_features, normdir, edge_index,
           edge_weight, edge_attr, W1, b1, W2, b2, Wf, bf, Wm1, bm1, Wm2, bm2,
           Wmix):
    xs = scalar_node_features[:, 0, :]
    xv = vector_node_features.reshape(N, 3 * H)
    src = edge_index[0]
    dst = edge_index[1]
    ew = edge_weight.reshape(E, 1)

    xxv = _node_tables(xs, xv, W1, b1.reshape(1, H), W2, b2.reshape(1, 3 * H))
    w_filt = _edge_filter(edge_attr, ew, Wf, bf.reshape(1, 3 * H))
    xxvj0 = _sc_gather(xxv, src[:E2])
    xxvj1 = _sc_gather(xxv, src[E2:])
    msg0 = _messages(w_filt, xxvj0, normdir, 0)
    msg1 = _messages(w_filt, xxvj1, normdir, 1)
    zeros = jnp.zeros((4, N, H), dtype=jnp.float32)
    agg0 = _sc_scatter(msg0, dst[:E2], zeros)
    agg = _sc_scatter(msg1, dst[E2:], agg0)
    s_out, v_out = _mixing(xs, xv, agg, Wmix, Wm1, bm1.reshape(1, H),
                           Wm2, bm2.reshape(1, 3 * H))
    return s_out.reshape(N, 1, H), v_out.reshape(N, 3, H)


# speedup vs baseline: 1.3004x; 1.2634x over previous
import functools

import jax
import jax.numpy as jnp
import numpy as np
from jax import lax
from jax.experimental import pallas as pl
from jax.experimental.pallas import tpu as pltpu
from jax.experimental.pallas import tpu_sc as plsc

N = 10000
E = 320000
H = 128
RBF = 16
CUTOFF = 5.0
EPS = 1e-8

NC = 2
NS = 16
WIN = 128
NWIN = E // WIN
BE = 3200
E2 = E // 2
NWIN2 = E2 // WIN
BN = 2000


def _node_tables_body(xs_ref, xv_ref, w1_ref, b1_ref, w2_ref, b2_ref,
                      out_ref):
    h = jnp.dot(xs_ref[...], w1_ref[...], preferred_element_type=jnp.float32)
    h = h + b1_ref[...]
    h = h * jax.nn.sigmoid(h)
    x = jnp.dot(h, w2_ref[...], preferred_element_type=jnp.float32) + b2_ref[...]
    xb = x.astype(jnp.bfloat16).astype(jnp.float32)
    xvb = xv_ref[...].astype(jnp.bfloat16).astype(jnp.float32)
    xi = jax.lax.bitcast_convert_type(xb, jnp.int32)
    xvi = jax.lax.bitcast_convert_type(xvb, jnp.int32)
    out_ref[...] = jax.lax.shift_right_logical(xi, 16) | (xvi & jnp.int32(-65536))


def _message_body(ea_ref, ew_ref, wf_ref, bf_ref, xxvj_ref, nd_ref, out_ref):
    ew = ew_ref[...]
    cut = 0.5 * (jnp.cos(ew * (np.pi / CUTOFF)) + 1.0)
    cut = cut * (ew < CUTOFF).astype(jnp.float32)
    w = jnp.dot(ea_ref[...], wf_ref[...], preferred_element_type=jnp.float32)
    w = (w + bf_ref[...]) * cut
    packed = xxvj_ref[...]
    xj = jax.lax.bitcast_convert_type(packed << 16, jnp.float32)
    xvj = jax.lax.bitcast_convert_type(packed & jnp.int32(-65536), jnp.float32)
    out_ref[0] = w[:, :H] * xj[:, :H]
    dmu_r = w[:, H:2 * H] * xj[:, H:2 * H]
    dmu_mu = w[:, 2 * H:] * xj[:, 2 * H:]
    for c in range(3):
        out_ref[1 + c] = (
            dmu_r * nd_ref[:, c:c + 1] + dmu_mu * xvj[:, c * H:(c + 1) * H]
        )


def _mixing_body(xs_ref, xv_ref, agg_ref, wmix_ref, wm1_ref, bm1_ref,
                 wm2_ref, bm2_ref, s_out_ref, v_out_ref):
    s = xs_ref[...] + agg_ref[0]
    v = [xv_ref[:, c * H:(c + 1) * H] + agg_ref[1 + c] for c in range(3)]
    mm = [jnp.dot(v[c], wmix_ref[...], preferred_element_type=jnp.float32)
          for c in range(3)]
    mu_v = [m[:, :H] for m in mm]
    mu_w = [m[:, H:] for m in mm]
    mu_vn = jnp.sqrt(mu_v[0] ** 2 + mu_v[1] ** 2 + mu_v[2] ** 2 + EPS)
    ctx = jnp.concatenate([s, mu_vn], axis=-1)
    h = jnp.dot(ctx, wm1_ref[...], preferred_element_type=jnp.float32) + bm1_ref[...]
    h = h * jax.nn.sigmoid(h)
    xc = jnp.dot(h, wm2_ref[...], preferred_element_type=jnp.float32) + bm2_ref[...]
    dq_i = xc[:, :H]
    dmu_i = xc[:, H:2 * H]
    dqmu_i = xc[:, 2 * H:]
    sum_vw = mu_v[0] * mu_w[0] + mu_v[1] * mu_w[1] + mu_v[2] * mu_w[2]
    s_out_ref[...] = s + dq_i + dqmu_i * sum_vw
    v_out_ref[...] = jnp.concatenate(
        [v[c] + dmu_i * mu_w[c] for c in range(3)], axis=-1)


def _node_tables(xs, xv, w1, b1, w2, b2):
    return pl.pallas_call(
        _node_tables_body,
        grid=(N // BN,),
        in_specs=[
            pl.BlockSpec((BN, H), lambda i: (i, 0)),
            pl.BlockSpec((BN, 3 * H), lambda i: (i, 0)),
            pl.BlockSpec((H, H), lambda i: (0, 0)),
            pl.BlockSpec((1, H), lambda i: (0, 0)),
            pl.BlockSpec((H, 3 * H), lambda i: (0, 0)),
            pl.BlockSpec((1, 3 * H), lambda i: (0, 0)),
        ],
        out_specs=pl.BlockSpec((BN, 3 * H), lambda i: (i, 0)),
        out_shape=jax.ShapeDtypeStruct((N, 3 * H), jnp.int32),
    )(xs, xv, w1, b1, w2, b2)


def _messages(ea, ew, wf, bf, xxvj, nd, half):
    off = half * (E2 // BE)
    return pl.pallas_call(
        _message_body,
        grid=(E2 // BE,),
        in_specs=[
            pl.BlockSpec((BE, RBF), lambda i: (i + off, 0)),
            pl.BlockSpec((BE, 1), lambda i: (i + off, 0)),
            pl.BlockSpec((RBF, 3 * H), lambda i: (0, 0)),
            pl.BlockSpec((1, 3 * H), lambda i: (0, 0)),
            pl.BlockSpec((BE, 3 * H), lambda i: (i, 0)),
            pl.BlockSpec((BE, 3), lambda i: (i + off, 0)),
        ],
        out_specs=pl.BlockSpec((4, BE, H), lambda i: (0, i, 0)),
        out_shape=jax.ShapeDtypeStruct((4, E2, H), jnp.float32),
    )(ea, ew, wf, bf, xxvj, nd)


def _mixing(xs, xv, agg, wmix, wm1, bm1, wm2, bm2):
    return pl.pallas_call(
        _mixing_body,
        grid=(N // BN,),
        in_specs=[
            pl.BlockSpec((BN, H), lambda i: (i, 0)),
            pl.BlockSpec((BN, 3 * H), lambda i: (i, 0)),
            pl.BlockSpec((4, BN, H), lambda i: (0, i, 0)),
            pl.BlockSpec((H, 2 * H), lambda i: (0, 0)),
            pl.BlockSpec((2 * H, H), lambda i: (0, 0)),
            pl.BlockSpec((1, H), lambda i: (0, 0)),
            pl.BlockSpec((H, 3 * H), lambda i: (0, 0)),
            pl.BlockSpec((1, 3 * H), lambda i: (0, 0)),
        ],
        out_specs=[
            pl.BlockSpec((BN, H), lambda i: (i, 0)),
            pl.BlockSpec((BN, 3 * H), lambda i: (i, 0)),
        ],
        out_shape=[
            jax.ShapeDtypeStruct((N, H), jnp.float32),
            jax.ShapeDtypeStruct((N, 3 * H), jnp.float32),
        ],
    )(xs, xv, agg, wmix, wm1, bm1, wm2, bm2)


def _sc_mesh():
    return plsc.VectorSubcoreMesh(
        core_axis_name="c", subcore_axis_name="s", num_cores=NC, num_subcores=NS)


def _sc_gather_body(tab_hbm, src_hbm, out_hbm,
                    idx0, idx1, gb0, gb1, si0, si1, sg0, sg1, sw0, sw1):
    wid = lax.axis_index("s") * NC + lax.axis_index("c")
    nw = NC * NS
    idxb = (idx0, idx1)
    gbuf = (gb0, gb1)
    si = (si0, si1)
    sg = (sg0, sg1)
    sw = (sw0, sw1)

    def start_idx(w, s):
        pltpu.async_copy(src_hbm.at[pl.ds(w * WIN, WIN)], idxb[s], si[s])

    def wait_idx(s):
        pltpu.make_async_copy(src_hbm.at[pl.ds(0, WIN)], idxb[s], si[s]).wait()

    def start_g(s):
        pltpu.async_copy(tab_hbm.at[idxb[s]], gbuf[s], sg[s])

    def wait_g(s):
        pltpu.make_async_copy(tab_hbm.at[idxb[s]], gbuf[s], sg[s]).wait()

    def start_wr(w, s):
        pltpu.async_copy(gbuf[s], out_hbm.at[pl.ds(w * WIN, WIN)], sw[s])

    def wait_wr(s):
        pltpu.make_async_copy(gbuf[s], out_hbm.at[pl.ds(0, WIN)], sw[s]).wait()

    start_idx(wid, 0)
    start_idx(wid + nw, 1)
    wait_idx(0)
    start_g(0)

    @pl.loop(0, NWIN2 // (2 * NC * NS) + 2)
    def _(p):
        for r in range(2):
            k = p * 2 + r
            w = wid + nw * k
            s = r
            s1 = (r + 1) % 2

            @pl.when(w < NWIN2)
            def _():
                wait_g(s)

            @pl.when(jnp.logical_and(k >= 1, w - nw < NWIN2))
            def _():
                wait_wr(s1)

            @pl.when(w + nw < NWIN2)
            def _():
                wait_idx(s1)
                start_g(s1)

            @pl.when(w < NWIN2)
            def _():
                start_wr(w, s)

            @pl.when(w + 2 * nw < NWIN2)
            def _():
                start_idx(w + 2 * nw, s)


def _sc_gather(tab, srcidx):
    k = pl.kernel(
        _sc_gather_body,
        out_type=jax.ShapeDtypeStruct((E2, 3 * H), jnp.int32),
        mesh=_sc_mesh(),
        scratch_types=(
            [pltpu.VMEM((WIN,), jnp.int32)] * 2
            + [pltpu.VMEM((WIN, 3 * H), jnp.int32)] * 2
            + [pltpu.SemaphoreType.DMA] * 6
        ),
    )
    return k(tab, srcidx)


def _sc_scatter_body(msg_hbm, dst_hbm, init_hbm, out_hbm, idxb, msgb, acc):
    cid = lax.axis_index("c")
    sid = lax.axis_index("s")
    iters = (NWIN2 + NS - 1) // NS

    for j in range(2):
        chunk = cid * 2 + j

        @pl.when(sid == 0)
        def _():
            pltpu.sync_copy(init_hbm.at[chunk], acc)

        plsc.subcore_barrier()

        @pl.loop(0, iters)
        def _(k):
            w = sid + NS * k

            @pl.when(w < NWIN2)
            def _():
                base = w * WIN
                pltpu.sync_copy(dst_hbm.at[pl.ds(base, WIN)], idxb)
                pltpu.sync_copy(msg_hbm.at[chunk].at[pl.ds(base, WIN)], msgb)
                pltpu.sync_copy(msgb, acc.at[idxb], add=True)

        plsc.subcore_barrier()

        @pl.when(sid < NS - 1)
        def _():
            pltpu.sync_copy(
                acc.at[pl.ds(sid * 640, 640)],
                out_hbm.at[chunk].at[pl.ds(sid * 640, 640)])

        @pl.when(sid == NS - 1)
        def _():
            pltpu.sync_copy(
                acc.at[pl.ds(9600, N - 9600)],
                out_hbm.at[chunk].at[pl.ds(9600, N - 9600)])

        plsc.subcore_barrier()


def _sc_scatter(msg, dst, init):
    k = pl.kernel(
        _sc_scatter_body,
        out_type=jax.ShapeDtypeStruct((4, N, H), jnp.float32),
        mesh=_sc_mesh(),
        scratch_types=[
            pltpu.VMEM((WIN,), jnp.int32),
            pltpu.VMEM((WIN, H), jnp.float32),
            pltpu.VMEM_SHARED((N, H), jnp.float32),
        ],
    )
    return k(msg, dst, init)


def kernel(scalar_node_features, vector_node_features, normdir, edge_index,
           edge_weight, edge_attr, W1, b1, W2, b2, Wf, bf, Wm1, bm1, Wm2, bm2,
           Wmix):
    xs = scalar_node_features[:, 0, :]
    xv = vector_node_features.reshape(N, 3 * H)
    src = edge_index[0]
    dst = edge_index[1]
    ew = edge_weight.reshape(E, 1)

    xxv = _node_tables(xs, xv, W1, b1.reshape(1, H), W2, b2.reshape(1, 3 * H))
    bfr = bf.reshape(1, 3 * H)
    xxvj0 = _sc_gather(xxv, src[:E2])
    xxvj1 = _sc_gather(xxv, src[E2:])
    msg0 = _messages(edge_attr, ew, Wf, bfr, xxvj0, normdir, 0)
    msg1 = _messages(edge_attr, ew, Wf, bfr, xxvj1, normdir, 1)
    zeros = jnp.zeros((4, N, H), dtype=jnp.float32)
    agg0 = _sc_scatter(msg0, dst[:E2], zeros)
    agg = _sc_scatter(msg1, dst[E2:], agg0)
    s_out, v_out = _mixing(xs, xv, agg, Wmix, Wm1, bm1.reshape(1, H),
                           Wm2, bm2.reshape(1, 3 * H))
    return s_out.reshape(N, 1, H), v_out.reshape(N, 3, H)
